# Initial kernel scaffold; baseline (speedup 1.0000x reference)
import jax
import jax.numpy as jnp
from jax.experimental import pallas as pl


def kernel(x, edge_index, W1, a1_src, a1_dst, b1, W2, a2_src, a2_dst, b2):
    raise NotImplementedError("write your pallas kernel here")


# speedup vs baseline: 2.0554x; 2.0554x over previous
import functools

import jax
import jax.numpy as jnp
from jax.experimental import pallas as pl
from jax.experimental.pallas import tpu as pltpu

_NB = 1024
_EB = 640


def _dense_kernel(x_ref, b_ref, w_ref, asrc_ref, adst_ref,
                  h_ref, as_ref, ad_ref, *, H, C, elu_in):
    xb = x_ref[...] + b_ref[...]
    if elu_in:
        xb = jnp.where(xb > 0, xb, jnp.exp(xb) - 1.0)
    h = jax.lax.dot_general(xb, w_ref[...], (((1,), (0,)), ((), ())),
                            preferred_element_type=jnp.float32)
    h_ref[...] = h
    hh = h.reshape(h.shape[0], H, C)
    as_ref[...] = jnp.sum(hh * asrc_ref[...][None], axis=-1)
    ad_ref[...] = jnp.sum(hh * adst_ref[...][None], axis=-1)


def _edge_kernel(src_ref, dst_ref, as_ref, ad_ref, h_ref,
                 p_ref, g_ref, e_acc, g_acc, *, NB, num_nb):
    k = pl.program_id(1)

    @pl.when(k == 0)
    def _():
        e_acc[...] = jnp.zeros_like(e_acc)
        g_acc[...] = jnp.zeros_like(g_acc)

    src = src_ref[0]
    dst = dst_ref[0]
    rows = jax.lax.broadcasted_iota(jnp.int32, (NB, src.shape[-1]), 0) + k * NB
    st = (rows == src).astype(jnp.float32)
    dt = (rows == dst).astype(jnp.float32)
    cn = (((0,), (0,)), ((), ()))
    e_acc[...] += (
        jax.lax.dot_general(st, as_ref[...], cn, preferred_element_type=jnp.float32)
        + jax.lax.dot_general(dt, ad_ref[...], cn, preferred_element_type=jnp.float32))
    g_acc[...] += jax.lax.dot_general(st, h_ref[...], cn,
                                      preferred_element_type=jnp.float32)

    @pl.when(k == num_nb - 1)
    def _():
        e = e_acc[...]
        e = jnp.where(e > 0, e, 0.2 * e)
        p_ref[...] = jnp.exp(e)
        g_ref[...] = g_acc[...]


def _scatter_kernel(dst_ref, p_ref, g_ref, as_ref, ad_ref, h_ref, b_ref,
                    out_ref, out_acc, den_acc, *, ND, H, C, num_eb, mean_out):
    i = pl.program_id(0)
    j = pl.program_id(1)

    @pl.when(j == 0)
    def _():
        out_acc[...] = jnp.zeros_like(out_acc)
        den_acc[...] = jnp.zeros_like(den_acc)

    dst = dst_ref[0]
    rows = jax.lax.broadcasted_iota(jnp.int32, (ND, dst.shape[-1]), 0) + i * ND
    dt = (rows == dst).astype(jnp.float32)
    p = p_ref[...]
    cn = (((1,), (0,)), ((), ()))
    den_acc[...] += jax.lax.dot_general(dt, p, cn,
                                        preferred_element_type=jnp.float32)
    g = g_ref[...]
    m = (g.reshape(g.shape[0], H, C) * p[..., None]).reshape(g.shape[0], H * C)
    out_acc[...] += jax.lax.dot_general(dt, m, cn,
                                        preferred_element_type=jnp.float32)

    @pl.when(j == num_eb - 1)
    def _():
        e_self = as_ref[...] + ad_ref[...]
        e_self = jnp.where(e_self > 0, e_self, 0.2 * e_self)
        p_self = jnp.exp(e_self)
        den = den_acc[...] + p_self
        hb = h_ref[...].reshape(ND, H, C)
        out3 = (out_acc[...].reshape(ND, H, C) + p_self[..., None] * hb)
        out3 = out3 / (den[..., None] + 1e-16)
        if mean_out:
            out_ref[...] = jnp.mean(out3, axis=1) + b_ref[...]
        else:
            out_ref[...] = out3.reshape(ND, H * C) + b_ref[...]


def _gat_layer(xp, b_in, src3, dst3, W, a_src, a_dst, b_out, *,
               H, C, elu_in, mean_out):
    NP, IN = xp.shape
    HC = H * C
    EPb = src3.shape[0]
    NPb = NP // _NB
    EB = src3.shape[2]

    h, a_s, a_d = pl.pallas_call(
        functools.partial(_dense_kernel, H=H, C=C, elu_in=elu_in),
        grid=(NPb,),
        in_specs=[
            pl.BlockSpec((_NB, IN), lambda i: (i, 0)),
            pl.BlockSpec((1, IN), lambda i: (0, 0)),
            pl.BlockSpec((IN, HC), lambda i: (0, 0)),
            pl.BlockSpec((H, C), lambda i: (0, 0)),
            pl.BlockSpec((H, C), lambda i: (0, 0)),
        ],
        out_specs=[
            pl.BlockSpec((_NB, HC), lambda i: (i, 0)),
            pl.BlockSpec((_NB, H), lambda i: (i, 0)),
            pl.BlockSpec((_NB, H), lambda i: (i, 0)),
        ],
        out_shape=[
            jax.ShapeDtypeStruct((NP, HC), jnp.float32),
            jax.ShapeDtypeStruct((NP, H), jnp.float32),
            jax.ShapeDtypeStruct((NP, H), jnp.float32),
        ],
    )(xp, b_in, W, a_src, a_dst)

    EP = EPb * EB
    p, g = pl.pallas_call(
        functools.partial(_edge_kernel, NB=_NB, num_nb=NPb),
        grid=(EPb, NPb),
        in_specs=[
            pl.BlockSpec((1, 1, EB), lambda j, k: (j, 0, 0)),
            pl.BlockSpec((1, 1, EB), lambda j, k: (j, 0, 0)),
            pl.BlockSpec((_NB, H), lambda j, k: (k, 0)),
            pl.BlockSpec((_NB, H), lambda j, k: (k, 0)),
            pl.BlockSpec((_NB, HC), lambda j, k: (k, 0)),
        ],
        out_specs=[
            pl.BlockSpec((EB, H), lambda j, k: (j, 0)),
            pl.BlockSpec((EB, HC), lambda j, k: (j, 0)),
        ],
        out_shape=[
            jax.ShapeDtypeStruct((EP, H), jnp.float32),
            jax.ShapeDtypeStruct((EP, HC), jnp.float32),
        ],
        scratch_shapes=[
            pltpu.VMEM((EB, H), jnp.float32),
            pltpu.VMEM((EB, HC), jnp.float32),
        ],
    )(src3, dst3, a_s, a_d, h)

    width = C if mean_out else HC
    out = pl.pallas_call(
        functools.partial(_scatter_kernel, ND=_NB, H=H, C=C,
                          num_eb=EPb, mean_out=mean_out),
        grid=(NPb, EPb),
        in_specs=[
            pl.BlockSpec((1, 1, EB), lambda i, j: (j, 0, 0)),
            pl.BlockSpec((EB, H), lambda i, j: (j, 0)),
            pl.BlockSpec((EB, HC), lambda i, j: (j, 0)),
            pl.BlockSpec((_NB, H), lambda i, j: (i, 0)),
            pl.BlockSpec((_NB, H), lambda i, j: (i, 0)),
            pl.BlockSpec((_NB, HC), lambda i, j: (i, 0)),
            pl.BlockSpec((1, width), lambda i, j: (0, 0)),
        ],
        out_specs=pl.BlockSpec((_NB, width), lambda i, j: (i, 0)),
        out_shape=jax.ShapeDtypeStruct((NP, width), jnp.float32),
        scratch_shapes=[
            pltpu.VMEM((_NB, HC), jnp.float32),
            pltpu.VMEM((_NB, H), jnp.float32),
        ],
    )(dst3, p, g, a_s, a_d, h, b_out)
    return out


def kernel(x, edge_index, W1, a1_src, a1_dst, b1, W2, a2_src, a2_dst, b2):
    N, IN = x.shape
    E = edge_index.shape[1]
    H, C = a1_src.shape
    HC = H * C

    NP = ((N + _NB - 1) // _NB) * _NB
    EP = ((E + _EB - 1) // _EB) * _EB

    xp = jnp.pad(x.astype(jnp.float32), ((0, NP - N), (0, 0)))
    ei = edge_index.astype(jnp.int32)
    src = jnp.pad(ei[0], (0, EP - E), constant_values=0)
    dst = jnp.pad(ei[1], (0, EP - E), constant_values=NP)
    src3 = src.reshape(EP // _EB, 1, _EB)
    dst3 = dst.reshape(EP // _EB, 1, _EB)

    zeros_in = jnp.zeros((1, IN), jnp.float32)
    zeros_hc = jnp.zeros((1, HC), jnp.float32)

    out1 = _gat_layer(xp, zeros_in, src3, dst3, W1, a1_src, a1_dst, zeros_hc,
                      H=H, C=C, elu_in=False, mean_out=False)
    out2 = _gat_layer(out1, b1.reshape(1, HC), src3, dst3, W2, a2_src, a2_dst,
                      b2.reshape(1, C), H=H, C=C, elu_in=True, mean_out=True)
    return out2[:N]


# speedup vs baseline: 2.0778x; 1.0109x over previous
import functools

import jax
import jax.numpy as jnp
from jax.experimental import pallas as pl
from jax.experimental.pallas import tpu as pltpu

_NB = 1024
_EB = 640


def _dense_kernel(x_ref, b_ref, w_ref, asrc_ref, adst_ref,
                  h_ref, as_ref, ad_ref, *, H, C, elu_in):
    xb = x_ref[...] + b_ref[...]
    if elu_in:
        xb = jnp.where(xb > 0, xb, jnp.exp(xb) - 1.0)
    h = jax.lax.dot_general(xb, w_ref[...], (((1,), (0,)), ((), ())),
                            preferred_element_type=jnp.float32)
    h_ref[...] = h
    hh = h.reshape(h.shape[0], H, C)
    as_ref[...] = jnp.sum(hh * asrc_ref[...][None], axis=-1)
    ad_ref[...] = jnp.sum(hh * adst_ref[...][None], axis=-1)


def _edge_kernel(src_ref, dst_ref, as_ref, ad_ref, h_ref,
                 p_ref, g_ref, e_acc, g_acc, *, NB, num_nb):
    k = pl.program_id(1)

    @pl.when(k == 0)
    def _():
        e_acc[...] = jnp.zeros_like(e_acc)
        g_acc[...] = jnp.zeros_like(g_acc)

    src = src_ref[0]
    dst = dst_ref[0]
    rows = jax.lax.broadcasted_iota(jnp.int32, (NB, src.shape[-1]), 0) + k * NB
    st = (rows == src).astype(jnp.bfloat16)
    dt = (rows == dst).astype(jnp.bfloat16)
    cn = (((0,), (0,)), ((), ()))
    e_acc[...] += (
        jax.lax.dot_general(st, as_ref[...].astype(jnp.bfloat16), cn,
                            preferred_element_type=jnp.float32)
        + jax.lax.dot_general(dt, ad_ref[...].astype(jnp.bfloat16), cn,
                              preferred_element_type=jnp.float32))
    g_acc[...] += jax.lax.dot_general(st, h_ref[...].astype(jnp.bfloat16), cn,
                                      preferred_element_type=jnp.float32)

    @pl.when(k == num_nb - 1)
    def _():
        e = e_acc[...]
        e = jnp.where(e > 0, e, 0.2 * e)
        p_ref[...] = jnp.exp(e)
        g_ref[...] = g_acc[...]


def _scatter_kernel(dst_ref, p_ref, g_ref, as_ref, ad_ref, h_ref, b_ref,
                    out_ref, out_acc, den_acc, *, ND, H, C, num_eb, mean_out):
    i = pl.program_id(0)
    j = pl.program_id(1)

    @pl.when(j == 0)
    def _():
        out_acc[...] = jnp.zeros_like(out_acc)
        den_acc[...] = jnp.zeros_like(den_acc)

    dst = dst_ref[0]
    rows = jax.lax.broadcasted_iota(jnp.int32, (ND, dst.shape[-1]), 0) + i * ND
    dt = (rows == dst).astype(jnp.bfloat16)
    p = p_ref[...]
    cn = (((1,), (0,)), ((), ()))
    den_acc[...] += jax.lax.dot_general(dt, p.astype(jnp.bfloat16), cn,
                                        preferred_element_type=jnp.float32)
    g = g_ref[...]
    m = (g.reshape(g.shape[0], H, C) * p[..., None]).reshape(g.shape[0], H * C)
    out_acc[...] += jax.lax.dot_general(dt, m.astype(jnp.bfloat16), cn,
                                        preferred_element_type=jnp.float32)

    @pl.when(j == num_eb - 1)
    def _():
        e_self = as_ref[...] + ad_ref[...]
        e_self = jnp.where(e_self > 0, e_self, 0.2 * e_self)
        p_self = jnp.exp(e_self)
        den = den_acc[...] + p_self
        hb = h_ref[...].reshape(ND, H, C)
        out3 = (out_acc[...].reshape(ND, H, C) + p_self[..., None] * hb)
        out3 = out3 / (den[..., None] + 1e-16)
        if mean_out:
            out_ref[...] = jnp.mean(out3, axis=1) + b_ref[...]
        else:
            out_ref[...] = out3.reshape(ND, H * C) + b_ref[...]


def _gat_layer(xp, b_in, src3, dst3, W, a_src, a_dst, b_out, *,
               H, C, elu_in, mean_out):
    NP, IN = xp.shape
    HC = H * C
    EPb = src3.shape[0]
    NPb = NP // _NB
    EB = src3.shape[2]

    h, a_s, a_d = pl.pallas_call(
        functools.partial(_dense_kernel, H=H, C=C, elu_in=elu_in),
        grid=(NPb,),
        in_specs=[
            pl.BlockSpec((_NB, IN), lambda i: (i, 0)),
            pl.BlockSpec((1, IN), lambda i: (0, 0)),
            pl.BlockSpec((IN, HC), lambda i: (0, 0)),
            pl.BlockSpec((H, C), lambda i: (0, 0)),
            pl.BlockSpec((H, C), lambda i: (0, 0)),
        ],
        out_specs=[
            pl.BlockSpec((_NB, HC), lambda i: (i, 0)),
            pl.BlockSpec((_NB, H), lambda i: (i, 0)),
            pl.BlockSpec((_NB, H), lambda i: (i, 0)),
        ],
        out_shape=[
            jax.ShapeDtypeStruct((NP, HC), jnp.float32),
            jax.ShapeDtypeStruct((NP, H), jnp.float32),
            jax.ShapeDtypeStruct((NP, H), jnp.float32),
        ],
    )(xp, b_in, W, a_src, a_dst)

    EP = EPb * EB
    p, g = pl.pallas_call(
        functools.partial(_edge_kernel, NB=_NB, num_nb=NPb),
        grid=(EPb, NPb),
        in_specs=[
            pl.BlockSpec((1, 1, EB), lambda j, k: (j, 0, 0)),
            pl.BlockSpec((1, 1, EB), lambda j, k: (j, 0, 0)),
            pl.BlockSpec((_NB, H), lambda j, k: (k, 0)),
            pl.BlockSpec((_NB, H), lambda j, k: (k, 0)),
            pl.BlockSpec((_NB, HC), lambda j, k: (k, 0)),
        ],
        out_specs=[
            pl.BlockSpec((EB, H), lambda j, k: (j, 0)),
            pl.BlockSpec((EB, HC), lambda j, k: (j, 0)),
        ],
        out_shape=[
            jax.ShapeDtypeStruct((EP, H), jnp.float32),
            jax.ShapeDtypeStruct((EP, HC), jnp.float32),
        ],
        scratch_shapes=[
            pltpu.VMEM((EB, H), jnp.float32),
            pltpu.VMEM((EB, HC), jnp.float32),
        ],
    )(src3, dst3, a_s, a_d, h)

    width = C if mean_out else HC
    out = pl.pallas_call(
        functools.partial(_scatter_kernel, ND=_NB, H=H, C=C,
                          num_eb=EPb, mean_out=mean_out),
        grid=(NPb, EPb),
        in_specs=[
            pl.BlockSpec((1, 1, EB), lambda i, j: (j, 0, 0)),
            pl.BlockSpec((EB, H), lambda i, j: (j, 0)),
            pl.BlockSpec((EB, HC), lambda i, j: (j, 0)),
            pl.BlockSpec((_NB, H), lambda i, j: (i, 0)),
            pl.BlockSpec((_NB, H), lambda i, j: (i, 0)),
            pl.BlockSpec((_NB, HC), lambda i, j: (i, 0)),
            pl.BlockSpec((1, width), lambda i, j: (0, 0)),
        ],
        out_specs=pl.BlockSpec((_NB, width), lambda i, j: (i, 0)),
        out_shape=jax.ShapeDtypeStruct((NP, width), jnp.float32),
        scratch_shapes=[
            pltpu.VMEM((_NB, HC), jnp.float32),
            pltpu.VMEM((_NB, H), jnp.float32),
        ],
    )(dst3, p, g, a_s, a_d, h, b_out)
    return out


def kernel(x, edge_index, W1, a1_src, a1_dst, b1, W2, a2_src, a2_dst, b2):
    N, IN = x.shape
    E = edge_index.shape[1]
    H, C = a1_src.shape
    HC = H * C

    NP = ((N + _NB - 1) // _NB) * _NB
    EP = ((E + _EB - 1) // _EB) * _EB

    xp = jnp.pad(x.astype(jnp.float32), ((0, NP - N), (0, 0)))
    ei = edge_index.astype(jnp.int32)
    src = jnp.pad(ei[0], (0, EP - E), constant_values=0)
    dst = jnp.pad(ei[1], (0, EP - E), constant_values=NP)
    src3 = src.reshape(EP // _EB, 1, _EB)
    dst3 = dst.reshape(EP // _EB, 1, _EB)

    zeros_in = jnp.zeros((1, IN), jnp.float32)
    zeros_hc = jnp.zeros((1, HC), jnp.float32)

    out1 = _gat_layer(xp, zeros_in, src3, dst3, W1, a1_src, a1_dst, zeros_hc,
                      H=H, C=C, elu_in=False, mean_out=False)
    out2 = _gat_layer(out1, b1.reshape(1, HC), src3, dst3, W2, a2_src, a2_dst,
                      b2.reshape(1, C), H=H, C=C, elu_in=True, mean_out=True)
    return out2[:N]


# speedup vs baseline: 2.2978x; 1.1059x over previous
import functools

import jax
import jax.numpy as jnp
from jax.experimental import pallas as pl
from jax.experimental.pallas import tpu as pltpu

_NB = 1024
_EB = 640


def _dense_kernel(x_ref, b_ref, w_ref, asrc_ref, adst_ref,
                  h_ref, hb_ref, as_ref, ad_ref, *, H, C, elu_in):
    xb = x_ref[...] + b_ref[...]
    if elu_in:
        xb = jnp.where(xb > 0, xb, jnp.exp(xb) - 1.0)
    h = jax.lax.dot_general(xb, w_ref[...], (((1,), (0,)), ((), ())),
                            preferred_element_type=jnp.float32)
    h_ref[...] = h
    hb_ref[...] = h.astype(jnp.bfloat16)
    hh = h.reshape(h.shape[0], H, C)
    as_ref[...] = jnp.sum(hh * asrc_ref[...][None], axis=-1)
    ad_ref[...] = jnp.sum(hh * adst_ref[...][None], axis=-1)


def _edge_kernel(src_ref, dst_ref, asrc_ref, ad_ref, hb_ref,
                 p_ref, g_ref, e_acc, g_acc, *, H, C, NB, num_nb):
    k = pl.program_id(1)

    @pl.when(k == 0)
    def _():
        e_acc[...] = jnp.zeros_like(e_acc)
        g_acc[...] = jnp.zeros_like(g_acc)

    src = src_ref[0]
    dst = dst_ref[0]
    rows = jax.lax.broadcasted_iota(jnp.int32, (NB, src.shape[-1]), 0) + k * NB
    st = (rows == src).astype(jnp.bfloat16)
    dt = (rows == dst).astype(jnp.bfloat16)
    cn = (((0,), (0,)), ((), ()))
    e_acc[...] += jax.lax.dot_general(dt, ad_ref[...].astype(jnp.bfloat16), cn,
                                      preferred_element_type=jnp.float32)
    g_acc[...] += jax.lax.dot_general(st, hb_ref[...], cn,
                                      preferred_element_type=jnp.float32)

    @pl.when(k == num_nb - 1)
    def _():
        g = g_acc[...]
        asg = jnp.sum(g.reshape(g.shape[0], H, C) * asrc_ref[...][None], axis=-1)
        e = asg + e_acc[...]
        e = jnp.where(e > 0, e, 0.2 * e)
        p_ref[...] = jnp.exp(e)
        g_ref[...] = g.astype(jnp.bfloat16)


def _scatter_kernel(dst_ref, p_ref, g_ref, as_ref, ad_ref, h_ref, b_ref,
                    out_ref, out_acc, den_acc, *, ND, H, C, num_eb, mean_out):
    i = pl.program_id(0)
    j = pl.program_id(1)

    @pl.when(j == 0)
    def _():
        out_acc[...] = jnp.zeros_like(out_acc)
        den_acc[...] = jnp.zeros_like(den_acc)

    dst = dst_ref[0]
    rows = jax.lax.broadcasted_iota(jnp.int32, (ND, dst.shape[-1]), 0) + i * ND
    dt = (rows == dst).astype(jnp.bfloat16)
    p = p_ref[...]
    pb = p.astype(jnp.bfloat16)
    cn = (((1,), (0,)), ((), ()))
    den_acc[...] += jax.lax.dot_general(dt, pb, cn,
                                        preferred_element_type=jnp.float32)
    g = g_ref[...]
    m = (g.reshape(g.shape[0], H, C) * pb[..., None]).reshape(g.shape[0], H * C)
    out_acc[...] += jax.lax.dot_general(dt, m, cn,
                                        preferred_element_type=jnp.float32)

    @pl.when(j == num_eb - 1)
    def _():
        e_self = as_ref[...] + ad_ref[...]
        e_self = jnp.where(e_self > 0, e_self, 0.2 * e_self)
        p_self = jnp.exp(e_self)
        den = den_acc[...] + p_self
        hb = h_ref[...].reshape(ND, H, C)
        out3 = (out_acc[...].reshape(ND, H, C) + p_self[..., None] * hb)
        out3 = out3 / (den[..., None] + 1e-16)
        if mean_out:
            out_ref[...] = jnp.mean(out3, axis=1) + b_ref[...]
        else:
            out_ref[...] = out3.reshape(ND, H * C) + b_ref[...]


def _gat_layer(xp, b_in, src3, dst3, W, a_src, a_dst, b_out, *,
               H, C, elu_in, mean_out):
    NP, IN = xp.shape
    HC = H * C
    EPb = src3.shape[0]
    NPb = NP // _NB
    EB = src3.shape[2]

    h, hb, a_s, a_d = pl.pallas_call(
        functools.partial(_dense_kernel, H=H, C=C, elu_in=elu_in),
        grid=(NPb,),
        in_specs=[
            pl.BlockSpec((_NB, IN), lambda i: (i, 0)),
            pl.BlockSpec((1, IN), lambda i: (0, 0)),
            pl.BlockSpec((IN, HC), lambda i: (0, 0)),
            pl.BlockSpec((H, C), lambda i: (0, 0)),
            pl.BlockSpec((H, C), lambda i: (0, 0)),
        ],
        out_specs=[
            pl.BlockSpec((_NB, HC), lambda i: (i, 0)),
            pl.BlockSpec((_NB, HC), lambda i: (i, 0)),
            pl.BlockSpec((_NB, H), lambda i: (i, 0)),
            pl.BlockSpec((_NB, H), lambda i: (i, 0)),
        ],
        out_shape=[
            jax.ShapeDtypeStruct((NP, HC), jnp.float32),
            jax.ShapeDtypeStruct((NP, HC), jnp.bfloat16),
            jax.ShapeDtypeStruct((NP, H), jnp.float32),
            jax.ShapeDtypeStruct((NP, H), jnp.float32),
        ],
    )(xp, b_in, W, a_src, a_dst)

    EP = EPb * EB
    p, g = pl.pallas_call(
        functools.partial(_edge_kernel, H=H, C=C, NB=_NB, num_nb=NPb),
        grid=(EPb, NPb),
        in_specs=[
            pl.BlockSpec((1, 1, EB), lambda j, k: (j, 0, 0)),
            pl.BlockSpec((1, 1, EB), lambda j, k: (j, 0, 0)),
            pl.BlockSpec((H, C), lambda j, k: (0, 0)),
            pl.BlockSpec((_NB, H), lambda j, k: (k, 0)),
            pl.BlockSpec((_NB, HC), lambda j, k: (k, 0)),
        ],
        out_specs=[
            pl.BlockSpec((EB, H), lambda j, k: (j, 0)),
            pl.BlockSpec((EB, HC), lambda j, k: (j, 0)),
        ],
        out_shape=[
            jax.ShapeDtypeStruct((EP, H), jnp.float32),
            jax.ShapeDtypeStruct((EP, HC), jnp.bfloat16),
        ],
        scratch_shapes=[
            pltpu.VMEM((EB, H), jnp.float32),
            pltpu.VMEM((EB, HC), jnp.float32),
        ],
    )(src3, dst3, a_src, a_d, hb)

    width = C if mean_out else HC
    out = pl.pallas_call(
        functools.partial(_scatter_kernel, ND=_NB, H=H, C=C,
                          num_eb=EPb, mean_out=mean_out),
        grid=(NPb, EPb),
        in_specs=[
            pl.BlockSpec((1, 1, EB), lambda i, j: (j, 0, 0)),
            pl.BlockSpec((EB, H), lambda i, j: (j, 0)),
            pl.BlockSpec((EB, HC), lambda i, j: (j, 0)),
            pl.BlockSpec((_NB, H), lambda i, j: (i, 0)),
            pl.BlockSpec((_NB, H), lambda i, j: (i, 0)),
            pl.BlockSpec((_NB, HC), lambda i, j: (i, 0)),
            pl.BlockSpec((1, width), lambda i, j: (0, 0)),
        ],
        out_specs=pl.BlockSpec((_NB, width), lambda i, j: (i, 0)),
        out_shape=jax.ShapeDtypeStruct((NP, width), jnp.float32),
        scratch_shapes=[
            pltpu.VMEM((_NB, HC), jnp.float32),
            pltpu.VMEM((_NB, H), jnp.float32),
        ],
    )(dst3, p, g, a_s, a_d, h, b_out)
    return out


def kernel(x, edge_index, W1, a1_src, a1_dst, b1, W2, a2_src, a2_dst, b2):
    N, IN = x.shape
    E = edge_index.shape[1]
    H, C = a1_src.shape
    HC = H * C

    NP = ((N + _NB - 1) // _NB) * _NB
    EP = ((E + _EB - 1) // _EB) * _EB

    xp = jnp.pad(x.astype(jnp.float32), ((0, NP - N), (0, 0)))
    ei = edge_index.astype(jnp.int32)
    src = jnp.pad(ei[0], (0, EP - E), constant_values=0)
    dst = jnp.pad(ei[1], (0, EP - E), constant_values=NP)
    src3 = src.reshape(EP // _EB, 1, _EB)
    dst3 = dst.reshape(EP // _EB, 1, _EB)

    zeros_in = jnp.zeros((1, IN), jnp.float32)
    zeros_hc = jnp.zeros((1, HC), jnp.float32)

    out1 = _gat_layer(xp, zeros_in, src3, dst3, W1, a1_src, a1_dst, zeros_hc,
                      H=H, C=C, elu_in=False, mean_out=False)
    out2 = _gat_layer(out1, b1.reshape(1, HC), src3, dst3, W2, a2_src, a2_dst,
                      b2.reshape(1, C), H=H, C=C, elu_in=True, mean_out=True)
    return out2[:N]


# speedup vs baseline: 2.3900x; 1.0401x over previous
import functools

import jax
import jax.numpy as jnp
from jax.experimental import pallas as pl
from jax.experimental.pallas import tpu as pltpu

_NB = 1024
_EB = 640


def _dense_kernel(x_ref, b_ref, w_ref, asrc_ref, adst_ref,
                  h_ref, hb_ref, as_ref, ad_ref, *, H, C, elu_in):
    xb = x_ref[...] + b_ref[...]
    if elu_in:
        xb = jnp.where(xb > 0, xb, jnp.exp(xb) - 1.0)
    h = jax.lax.dot_general(xb, w_ref[...], (((1,), (0,)), ((), ())),
                            preferred_element_type=jnp.float32)
    h_ref[...] = h
    hb_ref[...] = h.astype(jnp.bfloat16)
    hh = h.reshape(h.shape[0], H, C)
    as_ref[...] = jnp.sum(hh * asrc_ref[...][None], axis=-1)
    ad_ref[...] = jnp.sum(hh * adst_ref[...][None], axis=-1)


def _edge_kernel(src_ref, asrc_ref, hb_ref,
                 asg_ref, g_ref, g_acc, *, H, C, NB, num_nb):
    k = pl.program_id(1)

    @pl.when(k == 0)
    def _():
        g_acc[...] = jnp.zeros_like(g_acc)

    src = src_ref[0]
    rows = jax.lax.broadcasted_iota(jnp.int32, (NB, src.shape[-1]), 0) + k * NB
    st = (rows == src).astype(jnp.bfloat16)
    cn = (((0,), (0,)), ((), ()))
    g_acc[...] += jax.lax.dot_general(st, hb_ref[...], cn,
                                      preferred_element_type=jnp.float32)

    @pl.when(k == num_nb - 1)
    def _():
        g = g_acc[...]
        asg_ref[...] = jnp.sum(g.reshape(g.shape[0], H, C) * asrc_ref[...][None],
                               axis=-1)
        g_ref[...] = g.astype(jnp.bfloat16)


def _scatter_kernel(dst_ref, asg_ref, g_ref, as_ref, ad_ref, h_ref, b_ref,
                    out_ref, out_acc, den_acc, *, ND, H, C, num_eb, mean_out):
    i = pl.program_id(0)
    j = pl.program_id(1)

    @pl.when(j == 0)
    def _():
        out_acc[...] = jnp.zeros_like(out_acc)
        den_acc[...] = jnp.zeros_like(den_acc)

    dst = dst_ref[0]
    rows = jax.lax.broadcasted_iota(jnp.int32, (ND, dst.shape[-1]), 0) + i * ND
    dt = (rows == dst).astype(jnp.bfloat16)
    adg = jax.lax.dot_general(dt, ad_ref[...].astype(jnp.bfloat16),
                              (((0,), (0,)), ((), ())),
                              preferred_element_type=jnp.float32)
    e = asg_ref[...] + adg
    e = jnp.where(e > 0, e, 0.2 * e)
    pb = jnp.exp(e).astype(jnp.bfloat16)
    cn = (((1,), (0,)), ((), ()))
    den_acc[...] += jax.lax.dot_general(dt, pb, cn,
                                        preferred_element_type=jnp.float32)
    g = g_ref[...]
    m = (g.reshape(g.shape[0], H, C) * pb[..., None]).reshape(g.shape[0], H * C)
    out_acc[...] += jax.lax.dot_general(dt, m, cn,
                                        preferred_element_type=jnp.float32)

    @pl.when(j == num_eb - 1)
    def _():
        e_self = as_ref[...] + ad_ref[...]
        e_self = jnp.where(e_self > 0, e_self, 0.2 * e_self)
        p_self = jnp.exp(e_self)
        den = den_acc[...] + p_self
        hb = h_ref[...].reshape(ND, H, C)
        out3 = (out_acc[...].reshape(ND, H, C) + p_self[..., None] * hb)
        out3 = out3 / (den[..., None] + 1e-16)
        if mean_out:
            out_ref[...] = jnp.mean(out3, axis=1) + b_ref[...]
        else:
            out_ref[...] = out3.reshape(ND, H * C) + b_ref[...]


def _gat_layer(xp, b_in, src3, dst3, W, a_src, a_dst, b_out, *,
               H, C, elu_in, mean_out):
    NP, IN = xp.shape
    HC = H * C
    EPb = src3.shape[0]
    NPb = NP // _NB
    EB = src3.shape[2]

    h, hb, a_s, a_d = pl.pallas_call(
        functools.partial(_dense_kernel, H=H, C=C, elu_in=elu_in),
        grid=(NPb,),
        in_specs=[
            pl.BlockSpec((_NB, IN), lambda i: (i, 0)),
            pl.BlockSpec((1, IN), lambda i: (0, 0)),
            pl.BlockSpec((IN, HC), lambda i: (0, 0)),
            pl.BlockSpec((H, C), lambda i: (0, 0)),
            pl.BlockSpec((H, C), lambda i: (0, 0)),
        ],
        out_specs=[
            pl.BlockSpec((_NB, HC), lambda i: (i, 0)),
            pl.BlockSpec((_NB, HC), lambda i: (i, 0)),
            pl.BlockSpec((_NB, H), lambda i: (i, 0)),
            pl.BlockSpec((_NB, H), lambda i: (i, 0)),
        ],
        out_shape=[
            jax.ShapeDtypeStruct((NP, HC), jnp.float32),
            jax.ShapeDtypeStruct((NP, HC), jnp.bfloat16),
            jax.ShapeDtypeStruct((NP, H), jnp.float32),
            jax.ShapeDtypeStruct((NP, H), jnp.float32),
        ],
    )(xp, b_in, W, a_src, a_dst)

    EP = EPb * EB
    asg, g = pl.pallas_call(
        functools.partial(_edge_kernel, H=H, C=C, NB=_NB, num_nb=NPb),
        grid=(EPb, NPb),
        in_specs=[
            pl.BlockSpec((1, 1, EB), lambda j, k: (j, 0, 0)),
            pl.BlockSpec((H, C), lambda j, k: (0, 0)),
            pl.BlockSpec((_NB, HC), lambda j, k: (k, 0)),
        ],
        out_specs=[
            pl.BlockSpec((EB, H), lambda j, k: (j, 0)),
            pl.BlockSpec((EB, HC), lambda j, k: (j, 0)),
        ],
        out_shape=[
            jax.ShapeDtypeStruct((EP, H), jnp.float32),
            jax.ShapeDtypeStruct((EP, HC), jnp.bfloat16),
        ],
        scratch_shapes=[
            pltpu.VMEM((EB, HC), jnp.float32),
        ],
    )(src3, a_src, hb)

    width = C if mean_out else HC
    out = pl.pallas_call(
        functools.partial(_scatter_kernel, ND=_NB, H=H, C=C,
                          num_eb=EPb, mean_out=mean_out),
        grid=(NPb, EPb),
        in_specs=[
            pl.BlockSpec((1, 1, EB), lambda i, j: (j, 0, 0)),
            pl.BlockSpec((EB, H), lambda i, j: (j, 0)),
            pl.BlockSpec((EB, HC), lambda i, j: (j, 0)),
            pl.BlockSpec((_NB, H), lambda i, j: (i, 0)),
            pl.BlockSpec((_NB, H), lambda i, j: (i, 0)),
            pl.BlockSpec((_NB, HC), lambda i, j: (i, 0)),
            pl.BlockSpec((1, width), lambda i, j: (0, 0)),
        ],
        out_specs=pl.BlockSpec((_NB, width), lambda i, j: (i, 0)),
        out_shape=jax.ShapeDtypeStruct((NP, width), jnp.float32),
        scratch_shapes=[
            pltpu.VMEM((_NB, HC), jnp.float32),
            pltpu.VMEM((_NB, H), jnp.float32),
        ],
    )(dst3, asg, g, a_s, a_d, h, b_out)
    return out


def kernel(x, edge_index, W1, a1_src, a1_dst, b1, W2, a2_src, a2_dst, b2):
    N, IN = x.shape
    E = edge_index.shape[1]
    H, C = a1_src.shape
    HC = H * C

    NP = ((N + _NB - 1) // _NB) * _NB
    EP = ((E + _EB - 1) // _EB) * _EB

    xp = jnp.pad(x.astype(jnp.float32), ((0, NP - N), (0, 0)))
    ei = edge_index.astype(jnp.int32)
    src = jnp.pad(ei[0], (0, EP - E), constant_values=0)
    dst = jnp.pad(ei[1], (0, EP - E), constant_values=NP)
    src3 = src.reshape(EP // _EB, 1, _EB)
    dst3 = dst.reshape(EP // _EB, 1, _EB)

    zeros_in = jnp.zeros((1, IN), jnp.float32)
    zeros_hc = jnp.zeros((1, HC), jnp.float32)

    out1 = _gat_layer(xp, zeros_in, src3, dst3, W1, a1_src, a1_dst, zeros_hc,
                      H=H, C=C, elu_in=False, mean_out=False)
    out2 = _gat_layer(out1, b1.reshape(1, HC), src3, dst3, W2, a2_src, a2_dst,
                      b2.reshape(1, C), H=H, C=C, elu_in=True, mean_out=True)
    return out2[:N]
